# baseline (speedup 1.0000x reference)
import functools

import jax
import jax.numpy as jnp
from jax.experimental import pallas as pl
from jax.experimental.pallas import tpu as pltpu

_B, _C, _N = 8, 64, 100000
_TN = 25600
_NB = -(-_N // _TN)
_TOT = _B * _NB
_PARK = 8
_REF = _TOT - _PARK
_EPS = 1e-5
_CH = 2560
_NCH = _TN // _CH
_TAIL = _N - (_NB - 1) * _TN

_PIN_B, _PIN_N = (_REF - 1) // _NB, (_REF - 1) % _NB


def _fused_kernel(x_ref, w_ref, g_ref, b_ref, o_ref,
                  zs, s_acc, q_acc, w2_s, scale_s, shift_s):
    ph = pl.program_id(0)
    bi = pl.program_id(1)
    ni = pl.program_id(2)
    idx = bi * _NB + ni

    @pl.when((ph == 0) & (idx == 0))
    def _init():
        s_acc[...] = jnp.zeros_like(s_acc)
        q_acc[...] = jnp.zeros_like(q_acc)

    @pl.when(ph == 0)
    def _sweep():
        w = w_ref[...]
        park = idx >= _REF
        pidx = jnp.maximum(idx - _REF, 0)
        tail = ni == _NB - 1
        s = jnp.zeros((_C, 1), jnp.float32)
        q = jnp.zeros((_C, 1), jnp.float32)
        for k in range(_NCH):
            zc = jnp.dot(w, x_ref[0, :, k * _CH:(k + 1) * _CH],
                         preferred_element_type=jnp.float32)

            if k * _CH >= _TAIL:
                zm = jnp.where(tail, 0.0, zc)
            elif (k + 1) * _CH > _TAIL:
                col = jax.lax.broadcasted_iota(jnp.int32, (_C, _CH), 1)
                zm = jnp.where(tail & (col >= _TAIL - k * _CH), 0.0, zc)
            else:
                zm = zc
            s = s + jnp.sum(zm, axis=1, keepdims=True)
            q = q + jnp.sum(zm * zm, axis=1, keepdims=True)

            @pl.when(park)
            def _park():
                zs[pidx, :, k * _CH:(k + 1) * _CH] = zc.astype(jnp.bfloat16)

        s_acc[...] += s
        q_acc[...] += q

    @pl.when((ph == 0) & (idx == _TOT - 1))
    def _finish_stats():
        cnt = jnp.float32(_B * _N)
        mean = s_acc[...] / cnt
        var = q_acc[...] / cnt - mean * mean
        inv = g_ref[...] * jax.lax.rsqrt(var + _EPS)
        scale_s[...] = inv
        shift_s[...] = b_ref[...] - mean * inv
        w2_s[...] = w_ref[...] * inv

    @pl.when((ph == 1) & (idx < _REF))
    def _recompute():
        w2 = w2_s[...]
        sh = shift_s[...]
        for k in range(_NCH):
            zc = jnp.dot(w2, x_ref[0, :, k * _CH:(k + 1) * _CH],
                         preferred_element_type=jnp.float32)
            o_ref[0, :, k * _CH:(k + 1) * _CH] = jnp.maximum(zc + sh, 0.0)

    @pl.when((ph == 1) & (idx >= _REF))
    def _replay():
        pidx = jnp.maximum(idx - _REF, 0)
        sc = scale_s[...]
        sh = shift_s[...]
        for k in range(_NCH):
            zc = zs[pidx, :, k * _CH:(k + 1) * _CH].astype(jnp.float32)
            o_ref[0, :, k * _CH:(k + 1) * _CH] = jnp.maximum(
                zc * sc + sh, 0.0)


def _x_index_map(p, b, n):
    idx = b * _NB + n
    pinned = (p == 1) & (idx >= _REF)
    return (jnp.where(pinned, _PIN_B, b), 0, jnp.where(pinned, _PIN_N, n))


def _out_index_map(p, b, n):
    return (jnp.where(p == 0, 0, b), 0, jnp.where(p == 0, 0, n))


@functools.partial(jax.jit, static_argnames=())
def _run(x, W, gamma, beta):
    g2 = gamma.reshape(_C, 1)
    b2 = beta.reshape(_C, 1)

    y = pl.pallas_call(
        _fused_kernel,
        grid=(2, _B, _NB),
        in_specs=[
            pl.BlockSpec((1, _C, _TN), _x_index_map),
            pl.BlockSpec((_C, _C), lambda p, b, n: (0, 0)),
            pl.BlockSpec((_C, 1), lambda p, b, n: (0, 0)),
            pl.BlockSpec((_C, 1), lambda p, b, n: (0, 0)),
        ],
        out_specs=pl.BlockSpec((1, _C, _TN), _out_index_map),
        out_shape=jax.ShapeDtypeStruct((_B, _C, _N), jnp.float32),
        scratch_shapes=[
            pltpu.VMEM((_PARK, _C, _TN), jnp.bfloat16),
            pltpu.VMEM((_C, 1), jnp.float32),
            pltpu.VMEM((_C, 1), jnp.float32),
            pltpu.VMEM((_C, _C), jnp.float32),
            pltpu.VMEM((_C, 1), jnp.float32),
            pltpu.VMEM((_C, 1), jnp.float32),
        ],
        compiler_params=pltpu.CompilerParams(
            vmem_limit_bytes=64 * 1024 * 1024,
        ),
    )(x, W, g2, b2)

    return y


def kernel(p, x, W, gamma, beta):
    return (p, _run(x, W, gamma, beta))


# speedup vs baseline: 1.2061x; 1.2061x over previous
import functools

import jax
import jax.numpy as jnp
from jax.experimental import pallas as pl
from jax.experimental.pallas import tpu as pltpu

_B, _C, _N = 8, 64, 100000
_TN = 25600
_NB = -(-_N // _TN)
_TOT = _B * _NB
_PARK = 8
_REF = _TOT - _PARK
_EPS = 1e-5

_PIN_B, _PIN_N = (_REF - 1) // _NB, (_REF - 1) % _NB


def _fused_kernel(x_ref, w_ref, g_ref, b_ref, o_ref,
                  zs, c_acc, s_acc, sz_acc, q_acc, w2_s, scale_s, shift_s):
    ph = pl.program_id(0)
    bi = pl.program_id(1)
    ni = pl.program_id(2)
    idx = bi * _NB + ni

    @pl.when((ph == 0) & (idx == 0))
    def _init():
        c_acc[...] = jnp.zeros_like(c_acc)
        s_acc[...] = jnp.zeros_like(s_acc)
        sz_acc[...] = jnp.zeros_like(sz_acc)
        q_acc[...] = jnp.zeros_like(q_acc)

    @pl.when((ph == 0) & (idx < _REF) & (ni < _NB - 1))
    def _sweep_moment():
        xb = x_ref[0]
        c_acc[...] += jax.lax.dot_general(
            xb, xb, (((1,), (1,)), ((), ())),
            preferred_element_type=jnp.float32)
        s_acc[...] += jnp.sum(xb, axis=1, keepdims=True)

    @pl.when((ph == 0) & (idx < _REF) & (ni == _NB - 1))
    def _sweep_moment_tail():
        col = jax.lax.broadcasted_iota(jnp.int32, (_C, _TN), 1)
        xb = jnp.where(col < (_N - ni * _TN), x_ref[0], 0.0)
        c_acc[...] += jax.lax.dot_general(
            xb, xb, (((1,), (1,)), ((), ())),
            preferred_element_type=jnp.float32)
        s_acc[...] += jnp.sum(xb, axis=1, keepdims=True)

    @pl.when((ph == 0) & (idx >= _REF))
    def _sweep_park():
        z = jnp.dot(w_ref[...], x_ref[0], preferred_element_type=jnp.float32)
        zs[jnp.maximum(idx - _REF, 0)] = z.astype(jnp.bfloat16)

        @pl.when(ni < _NB - 1)
        def _stats_full():
            sz_acc[...] += jnp.sum(z, axis=1, keepdims=True)
            q_acc[...] += jnp.sum(z * z, axis=1, keepdims=True)

        @pl.when(ni == _NB - 1)
        def _stats_tail():
            col = jax.lax.broadcasted_iota(jnp.int32, (_C, _TN), 1)
            zm = jnp.where(col < (_N - ni * _TN), z, 0.0)
            sz_acc[...] += jnp.sum(zm, axis=1, keepdims=True)
            q_acc[...] += jnp.sum(zm * zm, axis=1, keepdims=True)

    @pl.when((ph == 0) & (idx == _TOT - 1))
    def _finish_stats():
        cnt = jnp.float32(_B * _N)
        w = w_ref[...]
        mean = (jnp.dot(w, s_acc[...], preferred_element_type=jnp.float32)
                + sz_acc[...]) / cnt
        a = jnp.dot(w, c_acc[...], preferred_element_type=jnp.float32)
        esq = (jnp.sum(a * w, axis=1, keepdims=True) + q_acc[...]) / cnt
        var = esq - mean * mean
        inv = g_ref[...] * jax.lax.rsqrt(var + _EPS)
        scale_s[...] = inv
        shift_s[...] = b_ref[...] - mean * inv
        w2_s[...] = w * inv

    @pl.when((ph == 1) & (idx < _REF))
    def _recompute():
        z = jnp.dot(w2_s[...], x_ref[0], preferred_element_type=jnp.float32)
        o_ref[0] = jnp.maximum(z + shift_s[...], 0.0)

    @pl.when((ph == 1) & (idx >= _REF))
    def _replay():
        z = zs[jnp.maximum(idx - _REF, 0)].astype(jnp.float32)
        o_ref[0] = jnp.maximum(z * scale_s[...] + shift_s[...], 0.0)


def _x_index_map(p, b, n):
    idx = b * _NB + n
    pinned = (p == 1) & (idx >= _REF)
    return (jnp.where(pinned, _PIN_B, b), 0, jnp.where(pinned, _PIN_N, n))


def _out_index_map(p, b, n):
    return (jnp.where(p == 0, 0, b), 0, jnp.where(p == 0, 0, n))


@functools.partial(jax.jit, static_argnames=())
def _run(x, W, gamma, beta):
    g2 = gamma.reshape(_C, 1)
    b2 = beta.reshape(_C, 1)

    y = pl.pallas_call(
        _fused_kernel,
        grid=(2, _B, _NB),
        in_specs=[
            pl.BlockSpec((1, _C, _TN), _x_index_map),
            pl.BlockSpec((_C, _C), lambda p, b, n: (0, 0)),
            pl.BlockSpec((_C, 1), lambda p, b, n: (0, 0)),
            pl.BlockSpec((_C, 1), lambda p, b, n: (0, 0)),
        ],
        out_specs=pl.BlockSpec((1, _C, _TN), _out_index_map),
        out_shape=jax.ShapeDtypeStruct((_B, _C, _N), jnp.float32),
        scratch_shapes=[
            pltpu.VMEM((_PARK, _C, _TN), jnp.bfloat16),
            pltpu.VMEM((_C, _C), jnp.float32),
            pltpu.VMEM((_C, 1), jnp.float32),
            pltpu.VMEM((_C, 1), jnp.float32),
            pltpu.VMEM((_C, 1), jnp.float32),
            pltpu.VMEM((_C, _C), jnp.float32),
            pltpu.VMEM((_C, 1), jnp.float32),
            pltpu.VMEM((_C, 1), jnp.float32),
        ],
        compiler_params=pltpu.CompilerParams(
            vmem_limit_bytes=64 * 1024 * 1024,
        ),
    )(x, W, g2, b2)

    return y


def kernel(p, x, W, gamma, beta):
    return (p, _run(x, W, gamma, beta))


# speedup vs baseline: 1.2414x; 1.0293x over previous
import functools

import jax
import jax.numpy as jnp
from jax.experimental import pallas as pl
from jax.experimental.pallas import tpu as pltpu

_B, _C, _N = 8, 64, 100000
_TN = 25600
_NB = -(-_N // _TN)
_TOT = _B * _NB
_PARK = 10
_REF = _TOT - _PARK
_EPS = 1e-5

_PIN_B, _PIN_N = (_REF - 1) // _NB, (_REF - 1) % _NB


def _fused_kernel(x_ref, w_ref, g_ref, b_ref, o_ref,
                  zs, c_acc, s_acc, sz_acc, q_acc, w2_s, scale_s, shift_s):
    ph = pl.program_id(0)
    bi = pl.program_id(1)
    ni = pl.program_id(2)
    idx = bi * _NB + ni

    @pl.when((ph == 0) & (idx == 0))
    def _init():
        c_acc[...] = jnp.zeros_like(c_acc)
        s_acc[...] = jnp.zeros_like(s_acc)
        sz_acc[...] = jnp.zeros_like(sz_acc)
        q_acc[...] = jnp.zeros_like(q_acc)

    @pl.when((ph == 0) & (idx < _REF) & (ni < _NB - 1))
    def _sweep_moment():
        xb = x_ref[0]
        c_acc[...] += jax.lax.dot_general(
            xb, xb, (((1,), (1,)), ((), ())),
            preferred_element_type=jnp.float32)
        s_acc[...] += jnp.sum(xb, axis=1, keepdims=True)

    @pl.when((ph == 0) & (idx < _REF) & (ni == _NB - 1))
    def _sweep_moment_tail():
        col = jax.lax.broadcasted_iota(jnp.int32, (_C, _TN), 1)
        xb = jnp.where(col < (_N - ni * _TN), x_ref[0], 0.0)
        c_acc[...] += jax.lax.dot_general(
            xb, xb, (((1,), (1,)), ((), ())),
            preferred_element_type=jnp.float32)
        s_acc[...] += jnp.sum(xb, axis=1, keepdims=True)

    @pl.when((ph == 0) & (idx >= _REF))
    def _sweep_park():
        pidx = jnp.maximum(idx - _REF, 0)
        _H = _TN // 2
        for h in range(2):
            zh = jnp.dot(w_ref[...], x_ref[0, :, h * _H:(h + 1) * _H],
                         preferred_element_type=jnp.float32)
            zs[pidx, :, h * _H:(h + 1) * _H] = zh.astype(jnp.bfloat16)
            if h == 0:
                sz_acc[...] += jnp.sum(zh, axis=1, keepdims=True)
                q_acc[...] += jnp.sum(zh * zh, axis=1, keepdims=True)
            else:
                lim = jnp.where(ni == _NB - 1, _N - ni * _TN - _H, _H)
                col = jax.lax.broadcasted_iota(jnp.int32, (_C, _H), 1)
                zm = jnp.where(col < lim, zh, 0.0)
                sz_acc[...] += jnp.sum(zm, axis=1, keepdims=True)
                q_acc[...] += jnp.sum(zm * zm, axis=1, keepdims=True)

    @pl.when((ph == 0) & (idx == _TOT - 1))
    def _finish_stats():
        cnt = jnp.float32(_B * _N)
        w = w_ref[...]
        mean = (jnp.dot(w, s_acc[...], preferred_element_type=jnp.float32)
                + sz_acc[...]) / cnt
        a = jnp.dot(w, c_acc[...], preferred_element_type=jnp.float32)
        esq = (jnp.sum(a * w, axis=1, keepdims=True) + q_acc[...]) / cnt
        var = esq - mean * mean
        inv = g_ref[...] * jax.lax.rsqrt(var + _EPS)
        scale_s[...] = inv
        shift_s[...] = b_ref[...] - mean * inv
        w2_s[...] = w * inv

    @pl.when((ph == 1) & (idx < _REF))
    def _recompute():
        _H = _TN // 2
        for h in range(2):
            zh = jnp.dot(w2_s[...], x_ref[0, :, h * _H:(h + 1) * _H],
                         preferred_element_type=jnp.float32)
            o_ref[0, :, h * _H:(h + 1) * _H] = jnp.maximum(
                zh + shift_s[...], 0.0)

    @pl.when((ph == 1) & (idx >= _REF))
    def _replay():
        z = zs[jnp.maximum(idx - _REF, 0)].astype(jnp.float32)
        o_ref[0] = jnp.maximum(z * scale_s[...] + shift_s[...], 0.0)


def _x_index_map(p, b, n):
    idx = b * _NB + n
    pinned = (p == 1) & (idx >= _REF)
    return (jnp.where(pinned, _PIN_B, b), 0, jnp.where(pinned, _PIN_N, n))


def _out_index_map(p, b, n):
    return (jnp.where(p == 0, 0, b), 0, jnp.where(p == 0, 0, n))


@functools.partial(jax.jit, static_argnames=())
def _run(x, W, gamma, beta):
    g2 = gamma.reshape(_C, 1)
    b2 = beta.reshape(_C, 1)

    y = pl.pallas_call(
        _fused_kernel,
        grid=(2, _B, _NB),
        in_specs=[
            pl.BlockSpec((1, _C, _TN), _x_index_map),
            pl.BlockSpec((_C, _C), lambda p, b, n: (0, 0)),
            pl.BlockSpec((_C, 1), lambda p, b, n: (0, 0)),
            pl.BlockSpec((_C, 1), lambda p, b, n: (0, 0)),
        ],
        out_specs=pl.BlockSpec((1, _C, _TN), _out_index_map),
        out_shape=jax.ShapeDtypeStruct((_B, _C, _N), jnp.float32),
        scratch_shapes=[
            pltpu.VMEM((_PARK, _C, _TN), jnp.bfloat16),
            pltpu.VMEM((_C, _C), jnp.float32),
            pltpu.VMEM((_C, 1), jnp.float32),
            pltpu.VMEM((_C, 1), jnp.float32),
            pltpu.VMEM((_C, 1), jnp.float32),
            pltpu.VMEM((_C, _C), jnp.float32),
            pltpu.VMEM((_C, 1), jnp.float32),
            pltpu.VMEM((_C, 1), jnp.float32),
        ],
        compiler_params=pltpu.CompilerParams(
            vmem_limit_bytes=64 * 1024 * 1024,
        ),
    )(x, W, g2, b2)

    return y


def kernel(p, x, W, gamma, beta):
    return (p, _run(x, W, gamma, beta))
